# baseline (speedup 1.0000x reference)
import functools

import jax
import jax.numpy as jnp
from jax import lax
from jax.experimental import pallas as pl
from jax.experimental.pallas import tpu as pltpu
from jax.experimental.pallas import tpu_sc as plsc

_IDX_CHUNK = 128
_STRIDE = 1 << 20


def _detile_body(ut_ref, it_ref, uo_ref, io_ref, sem):
    D, V = ut_ref.shape
    main = V - V % 128
    copies = []
    for d in range(D):
        copies.append(
            pltpu.make_async_copy(
                ut_ref.at[d, pl.ds(0, main)],
                uo_ref.at[pl.ds(d * _STRIDE, main)],
                sem,
            )
        )
        copies.append(
            pltpu.make_async_copy(
                it_ref.at[d, pl.ds(0, main)],
                io_ref.at[pl.ds(d * _STRIDE, main)],
                sem,
            )
        )
    for c in copies:
        c.start()
    for c in copies:
        c.wait()


@functools.lru_cache(maxsize=None)
def _build(B, V, D):
    info = plsc.get_sparse_core_info()
    NC, NS, L = info.num_cores, info.num_subcores, info.num_lanes
    NW = NC * NS
    assert B % NW == 0 and V <= _STRIDE
    b_per_w = B // NW
    n_chunks = b_per_w // _IDX_CHUNK
    main = V - V % 128
    tail = V - main
    mesh = plsc.VectorSubcoreMesh(core_axis_name="c", subcore_axis_name="s")

    flat = jax.ShapeDtypeStruct((D * _STRIDE,), jnp.float32)
    detile = pl.pallas_call(
        _detile_body,
        in_specs=[
            pl.BlockSpec(memory_space=pl.ANY),
            pl.BlockSpec(memory_space=pl.ANY),
        ],
        out_specs=[
            pl.BlockSpec(memory_space=pl.ANY),
            pl.BlockSpec(memory_space=pl.ANY),
        ],
        out_shape=[flat, flat],
        scratch_shapes=[pltpu.SemaphoreType.DMA],
    )

    @functools.partial(
        pl.kernel,
        mesh=mesh,
        out_type=jax.ShapeDtypeStruct((D * B,), jnp.float32),
        compiler_params=pltpu.CompilerParams(
            use_tc_tiling_on_sc=True, needs_layout_passes=False
        ),
        scratch_types=[
            pltpu.VMEM((b_per_w,), jnp.int32),
            pltpu.VMEM((b_per_w,), jnp.int32),
            pltpu.VMEM((b_per_w,), jnp.int32),
            pltpu.VMEM((b_per_w,), jnp.int32),
            pltpu.VMEM((b_per_w,), jnp.int32),
            pltpu.VMEM((b_per_w,), jnp.int32),
            pltpu.VMEM((D * b_per_w,), jnp.float32),
            pltpu.VMEM((D * b_per_w,), jnp.float32),
            pltpu.VMEM((D * tail,), jnp.float32),
            pltpu.VMEM((D * tail,), jnp.float32),
            pltpu.SemaphoreType.DMA,
            pltpu.SemaphoreType.DMA,
        ],
    )
    def gmf(uid_hbm, iid_hbm, up_hbm, ip_hbm, ut_hbm, it_hbm, out_hbm,
            uidx_v, iidx_v, utb_v, itb_v, umk_v, imk_v,
            uvals_v, ivals_v, utail_v, itail_v, sem_u, sem_i):
        wid = lax.axis_index("s") * NC + lax.axis_index("c")
        base = wid * b_per_w
        pltpu.sync_copy(uid_hbm.at[pl.ds(base, b_per_w)], uidx_v)
        pltpu.sync_copy(iid_hbm.at[pl.ds(base, b_per_w)], iidx_v)
        pltpu.sync_copy(ut_hbm, utail_v)
        pltpu.sync_copy(it_hbm, itail_v)

        zeros = jnp.zeros((L,), jnp.int32)
        ones = jnp.ones((L,), jnp.int32)

        def prep(g, carry):
            s = pl.ds(g * L, L)
            u = uidx_v[s]
            i = iidx_v[s]
            um = u >= main
            im = i >= main
            umk_v[s] = lax.select(um, ones, zeros)
            imk_v[s] = lax.select(im, ones, zeros)
            utb_v[s] = lax.select(um, u - main, zeros)
            itb_v[s] = lax.select(im, i - main, zeros)
            uidx_v[s] = lax.select(um, zeros, u)
            iidx_v[s] = lax.select(im, zeros, i)
            return carry

        lax.fori_loop(0, b_per_w // L, prep, 0)

        def fetch(d, carry):
            u_win = up_hbm.at[pl.ds(d * _STRIDE, _STRIDE)]
            i_win = ip_hbm.at[pl.ds(d * _STRIDE, _STRIDE)]
            for c in range(n_chunks):
                idx = pl.ds(c * _IDX_CHUNK, _IDX_CHUNK)
                dst = pl.ds(d * b_per_w + c * _IDX_CHUNK, _IDX_CHUNK)
                pltpu.async_copy(
                    u_win.at[uidx_v.at[idx]], uvals_v.at[dst], sem_u)
                pltpu.async_copy(
                    i_win.at[iidx_v.at[idx]], ivals_v.at[dst], sem_i)
            return carry

        lax.fori_loop(0, D, fetch, 0)
        pltpu.make_async_copy(
            up_hbm.at[pl.ds(0, D * b_per_w)], uvals_v, sem_u
        ).wait()
        pltpu.make_async_copy(
            ip_hbm.at[pl.ds(0, D * b_per_w)], ivals_v, sem_i
        ).wait()

        def merge_mul(d, carry):
            toff = d * tail
            for g in range(b_per_w // L):
                s = pl.ds(g * L, L)
                f = pl.ds(d * b_per_w + g * L, L)
                ut16 = plsc.load_gather(utail_v, [utb_v[s] + toff])
                it16 = plsc.load_gather(itail_v, [itb_v[s] + toff])
                u = lax.select(umk_v[s] > 0, ut16, uvals_v[f])
                i = lax.select(imk_v[s] > 0, it16, ivals_v[f])
                uvals_v[f] = u * i
            return carry

        lax.fori_loop(0, D, merge_mul, 0)

        def flush(d, carry):
            pltpu.async_copy(
                uvals_v.at[pl.ds(d * b_per_w, b_per_w)],
                out_hbm.at[pl.ds(d * B + base, b_per_w)],
                sem_u,
            )
            return carry

        lax.fori_loop(0, D, flush, 0)
        pltpu.make_async_copy(
            up_hbm.at[pl.ds(0, D * b_per_w)], uvals_v, sem_u
        ).wait()

    def run(user_ids, item_ids, user_table, item_table):
        ut = user_table.T
        it = item_table.T
        up, ip = detile(ut, it)
        tu = ut[:, main:].reshape(D * tail)
        ti = it[:, main:].reshape(D * tail)
        out1d = gmf(user_ids, item_ids, up, ip, tu, ti)
        return out1d.reshape(D, B).T

    return run


@jax.jit
def kernel(user_ids, item_ids, user_table, item_table):
    (B,) = user_ids.shape
    V, D = user_table.shape
    return _build(B, V, D)(user_ids, item_ids, user_table, item_table)


# speedup vs baseline: 21.4968x; 21.4968x over previous
import functools

import jax
import jax.numpy as jnp
from jax import lax
from jax.experimental import pallas as pl
from jax.experimental.pallas import tpu as pltpu
from jax.experimental.pallas import tpu_sc as plsc

_PACK = 4
_VB = 4096
_SH = _VB.bit_length() - 1
_WAVE = 128


def _detile_body(t_ref, out_ref):
    D = t_ref.shape[0]
    row = lax.broadcasted_iota(jnp.int32, (D, _PACK * D), 0)
    col = lax.broadcasted_iota(jnp.int32, (D, _PACK * D), 1)
    acc = None
    for t in range(_PACK):
        x_t = t_ref[:, t * _VB:(t + 1) * _VB]
        e_t = (col == t * D + row).astype(jnp.float32)
        z = lax.dot_general(
            x_t, e_t, (((0,), (0,)), ((), ())),
            precision=lax.Precision.DEFAULT,
            preferred_element_type=jnp.float32,
        )
        acc = z if acc is None else acc + z
    out_ref[...] = acc


@functools.lru_cache(maxsize=None)
def _build(B, V, D):
    info = plsc.get_sparse_core_info()
    NC, NS, L = info.num_cores, info.num_subcores, info.num_lanes
    NW = NC * NS
    assert B % NW == 0 and D == 32
    b_per_w = B // NW
    n_waves = b_per_w // _WAVE
    mesh = plsc.VectorSubcoreMesh(core_axis_name="c", subcore_axis_name="s")

    grid = (V + _PACK * _VB - 1) // (_PACK * _VB)
    packed_rows = grid * _VB
    in_specs = [pl.BlockSpec((D, _PACK * _VB), lambda c: (0, c))]
    detile = pl.pallas_call(
        _detile_body,
        grid=(grid,),
        in_specs=in_specs,
        out_specs=pl.BlockSpec((_VB, _PACK * D), lambda c: (c, 0)),
        out_shape=jax.ShapeDtypeStruct((packed_rows, _PACK * D), jnp.float32),
    )

    @functools.partial(
        pl.kernel,
        mesh=mesh,
        out_type=jax.ShapeDtypeStruct((D * B,), jnp.float32),
        compiler_params=pltpu.CompilerParams(
            use_tc_tiling_on_sc=True, needs_layout_passes=False
        ),
        scratch_types=[
            pltpu.VMEM((b_per_w,), jnp.int32),
            pltpu.VMEM((b_per_w,), jnp.int32),
            pltpu.VMEM((b_per_w,), jnp.int32),
            pltpu.VMEM((b_per_w,), jnp.int32),
            pltpu.VMEM((b_per_w,), jnp.int32),
            pltpu.VMEM((b_per_w,), jnp.int32),
            pltpu.VMEM((_WAVE, _PACK * D), jnp.float32),
            pltpu.VMEM((_WAVE, _PACK * D), jnp.float32),
            pltpu.VMEM((D * b_per_w,), jnp.float32),
            pltpu.SemaphoreType.DMA,
            pltpu.SemaphoreType.DMA,
            pltpu.SemaphoreType.DMA,
        ],
    )
    def gmf(uid_hbm, iid_hbm, up_hbm, ip_hbm, out_hbm,
            uids_v, iids_v, urb_v, irb_v, ulq_v, ilq_v,
            uw_v, iw_v, prod_v, sem_u, sem_i, sem_o):
        wid = lax.axis_index("s") * NC + lax.axis_index("c")
        base = wid * b_per_w
        pltpu.sync_copy(uid_hbm.at[pl.ds(base, b_per_w)], uids_v)
        pltpu.sync_copy(iid_hbm.at[pl.ds(base, b_per_w)], iids_v)

        def prep(g, carry):
            s = pl.ds(g * L, L)
            u = uids_v[s]
            i = iids_v[s]
            urb_v[s] = lax.shift_left(lax.shift_right_logical(u, _SH + 2), _SH) + \
                lax.bitwise_and(u, _VB - 1)
            irb_v[s] = lax.shift_left(lax.shift_right_logical(i, _SH + 2), _SH) + \
                lax.bitwise_and(i, _VB - 1)
            ulq_v[s] = lax.shift_left(
                lax.bitwise_and(lax.shift_right_logical(u, _SH), _PACK - 1), 5)
            ilq_v[s] = lax.shift_left(
                lax.bitwise_and(lax.shift_right_logical(i, _SH), _PACK - 1), 5)
            return carry

        lax.fori_loop(0, b_per_w // L, prep, 0)

        row_iota = lax.iota(jnp.int32, L)
        for w in range(n_waves):
            cu = pltpu.async_copy(
                up_hbm.at[urb_v.at[pl.ds(w * _WAVE, _WAVE)]], uw_v, sem_u)
            ci = pltpu.async_copy(
                ip_hbm.at[irb_v.at[pl.ds(w * _WAVE, _WAVE)]], iw_v, sem_i)
            cu.wait()
            ci.wait()

            def select(d, carry, _w=w):
                for g in range(_WAVE // L):
                    rows = row_iota + (g * L)
                    sl = pl.ds(_w * _WAVE + g * L, L)
                    uv = plsc.load_gather(uw_v, [rows, ulq_v[sl] + d])
                    iv = plsc.load_gather(iw_v, [rows, ilq_v[sl] + d])
                    prod_v[pl.ds(d * b_per_w + _w * _WAVE + g * L, L)] = uv * iv
                return carry

            lax.fori_loop(0, D, select, 0)

        def flush(d, carry):
            pltpu.async_copy(
                prod_v.at[pl.ds(d * b_per_w, b_per_w)],
                out_hbm.at[pl.ds(d * B + base, b_per_w)],
                sem_o,
            )
            return carry

        lax.fori_loop(0, D, flush, 0)
        pltpu.make_async_copy(
            out_hbm.at[pl.ds(0, D * b_per_w)], prod_v, sem_o
        ).wait()

    def run(user_ids, item_ids, user_table, item_table):
        up = detile(user_table.T)
        ip = detile(item_table.T)
        out1d = gmf(user_ids, item_ids, up, ip)
        return out1d.reshape(D, B).T

    return run


@jax.jit
def kernel(user_ids, item_ids, user_table, item_table):
    (B,) = user_ids.shape
    V, D = user_table.shape
    return _build(B, V, D)(user_ids, item_ids, user_table, item_table)


# speedup vs baseline: 22.7407x; 1.0579x over previous
import functools

import jax
import jax.numpy as jnp
from jax import lax
from jax.experimental import pallas as pl
from jax.experimental.pallas import tpu as pltpu
from jax.experimental.pallas import tpu_sc as plsc

_PACK = 4
_VB = 4096
_SH = _VB.bit_length() - 1
_WAVE = 128


def _detile_body(ut_ref, it_ref, uo_ref, io_ref):
    D = ut_ref.shape[0]
    row = lax.broadcasted_iota(jnp.int32, (D, _PACK * D), 0)
    col = lax.broadcasted_iota(jnp.int32, (D, _PACK * D), 1)
    for t_ref, out_ref in ((ut_ref, uo_ref), (it_ref, io_ref)):
        acc = None
        for t in range(_PACK):
            x_t = t_ref[:, t * _VB:(t + 1) * _VB]
            e_t = (col == t * D + row).astype(jnp.float32)
            z = lax.dot_general(
                x_t, e_t, (((0,), (0,)), ((), ())),
                precision=lax.Precision.DEFAULT,
                preferred_element_type=jnp.float32,
            )
            acc = z if acc is None else acc + z
        out_ref[...] = acc


@functools.lru_cache(maxsize=None)
def _build(B, V, D):
    info = plsc.get_sparse_core_info()
    NC, NS, L = info.num_cores, info.num_subcores, info.num_lanes
    NW = NC * NS
    assert B % NW == 0 and D == 32
    b_per_w = B // NW
    n_waves = b_per_w // _WAVE
    mesh = plsc.VectorSubcoreMesh(core_axis_name="c", subcore_axis_name="s")

    grid = (V + _PACK * _VB - 1) // (_PACK * _VB)
    packed_rows = grid * _VB
    packed = jax.ShapeDtypeStruct((packed_rows, _PACK * D), jnp.float32)
    detile = pl.pallas_call(
        _detile_body,
        grid=(grid,),
        in_specs=[
            pl.BlockSpec((D, _PACK * _VB), lambda c: (0, c)),
            pl.BlockSpec((D, _PACK * _VB), lambda c: (0, c)),
        ],
        out_specs=[
            pl.BlockSpec((_VB, _PACK * D), lambda c: (c, 0)),
            pl.BlockSpec((_VB, _PACK * D), lambda c: (c, 0)),
        ],
        out_shape=[packed, packed],
    )

    @functools.partial(
        pl.kernel,
        mesh=mesh,
        out_type=jax.ShapeDtypeStruct((D * B,), jnp.float32),
        compiler_params=pltpu.CompilerParams(
            use_tc_tiling_on_sc=True, needs_layout_passes=False
        ),
        scratch_types=[
            pltpu.VMEM((b_per_w,), jnp.int32),
            pltpu.VMEM((b_per_w,), jnp.int32),
            pltpu.VMEM((b_per_w,), jnp.int32),
            pltpu.VMEM((b_per_w,), jnp.int32),
            pltpu.VMEM((b_per_w,), jnp.int32),
            pltpu.VMEM((b_per_w,), jnp.int32),
            pltpu.VMEM((_WAVE, _PACK * D), jnp.float32),
            pltpu.VMEM((_WAVE, _PACK * D), jnp.float32),
            pltpu.VMEM((D * b_per_w,), jnp.float32),
            pltpu.SemaphoreType.DMA,
            pltpu.SemaphoreType.DMA,
            pltpu.SemaphoreType.DMA,
        ],
    )
    def gmf(uid_hbm, iid_hbm, up_hbm, ip_hbm, out_hbm,
            uids_v, iids_v, urb_v, irb_v, ulq_v, ilq_v,
            uw_v, iw_v, prod_v, sem_u, sem_i, sem_o):
        wid = lax.axis_index("s") * NC + lax.axis_index("c")
        base = wid * b_per_w
        pltpu.sync_copy(uid_hbm.at[pl.ds(base, b_per_w)], uids_v)
        pltpu.sync_copy(iid_hbm.at[pl.ds(base, b_per_w)], iids_v)

        def prep(g, carry):
            s = pl.ds(g * L, L)
            u = uids_v[s]
            i = iids_v[s]
            urb_v[s] = lax.shift_left(lax.shift_right_logical(u, _SH + 2), _SH) + \
                lax.bitwise_and(u, _VB - 1)
            irb_v[s] = lax.shift_left(lax.shift_right_logical(i, _SH + 2), _SH) + \
                lax.bitwise_and(i, _VB - 1)
            ulq_v[s] = lax.shift_left(
                lax.bitwise_and(lax.shift_right_logical(u, _SH), _PACK - 1), 5)
            ilq_v[s] = lax.shift_left(
                lax.bitwise_and(lax.shift_right_logical(i, _SH), _PACK - 1), 5)
            return carry

        lax.fori_loop(0, b_per_w // L, prep, 0)

        row_iota = lax.iota(jnp.int32, L)
        for w in range(n_waves):
            cu = pltpu.async_copy(
                up_hbm.at[urb_v.at[pl.ds(w * _WAVE, _WAVE)]], uw_v, sem_u)
            ci = pltpu.async_copy(
                ip_hbm.at[irb_v.at[pl.ds(w * _WAVE, _WAVE)]], iw_v, sem_i)
            cu.wait()
            ci.wait()

            def select(d, carry, _w=w):
                for g in range(_WAVE // L):
                    rows = row_iota + (g * L)
                    sl = pl.ds(_w * _WAVE + g * L, L)
                    uv = plsc.load_gather(uw_v, [rows, ulq_v[sl] + d])
                    iv = plsc.load_gather(iw_v, [rows, ilq_v[sl] + d])
                    prod_v[pl.ds(d * b_per_w + _w * _WAVE + g * L, L)] = uv * iv
                return carry

            lax.fori_loop(0, D, select, 0)

        def flush(d, carry):
            pltpu.async_copy(
                prod_v.at[pl.ds(d * b_per_w, b_per_w)],
                out_hbm.at[pl.ds(d * B + base, b_per_w)],
                sem_o,
            )
            return carry

        lax.fori_loop(0, D, flush, 0)
        pltpu.make_async_copy(
            out_hbm.at[pl.ds(0, D * b_per_w)], prod_v, sem_o
        ).wait()

    def run(user_ids, item_ids, user_table, item_table):
        up, ip = detile(user_table.T, item_table.T)
        out1d = gmf(user_ids, item_ids, up, ip)
        return out1d.reshape(D, B).T

    return run


@jax.jit
def kernel(user_ids, item_ids, user_table, item_table):
    (B,) = user_ids.shape
    V, D = user_table.shape
    return _build(B, V, D)(user_ids, item_ids, user_table, item_table)


# speedup vs baseline: 34.8072x; 1.5306x over previous
import functools

import jax
import jax.numpy as jnp
from jax import lax
from jax.experimental import pallas as pl
from jax.experimental.pallas import tpu as pltpu
from jax.experimental.pallas import tpu_sc as plsc

_PACK = 4
_VB = 4096
_SH = _VB.bit_length() - 1
_WAVE = 128


def _detile_body(ut_ref, it_ref, uo_ref, io_ref):
    D = ut_ref.shape[0]
    row = lax.broadcasted_iota(jnp.int32, (_PACK * D, _PACK * D), 0)
    col = lax.broadcasted_iota(jnp.int32, (_PACK * D, _PACK * D), 1)
    eye = (row == col).astype(jnp.float32)
    for t_ref, out_ref in ((ut_ref, uo_ref), (it_ref, io_ref)):
        v = jnp.concatenate(
            [t_ref[:, t * _VB:(t + 1) * _VB] for t in range(_PACK)], axis=0)
        out_ref[...] = lax.dot_general(
            v, eye, (((0,), (0,)), ((), ())),
            precision=lax.Precision.DEFAULT,
            preferred_element_type=jnp.float32,
        )


@functools.lru_cache(maxsize=None)
def _build(B, V, D):
    info = plsc.get_sparse_core_info()
    NC, NS, L = info.num_cores, info.num_subcores, info.num_lanes
    NW = NC * NS
    assert B % NW == 0 and D == 32
    b_per_w = B // NW
    n_waves = b_per_w // _WAVE
    mesh = plsc.VectorSubcoreMesh(core_axis_name="c", subcore_axis_name="s")

    grid = (V + _PACK * _VB - 1) // (_PACK * _VB)
    packed_rows = grid * _VB
    packed = jax.ShapeDtypeStruct((packed_rows, _PACK * D), jnp.float32)
    detile = pl.pallas_call(
        _detile_body,
        grid=(grid,),
        in_specs=[
            pl.BlockSpec((D, _PACK * _VB), lambda c: (0, c)),
            pl.BlockSpec((D, _PACK * _VB), lambda c: (0, c)),
        ],
        out_specs=[
            pl.BlockSpec((_VB, _PACK * D), lambda c: (c, 0)),
            pl.BlockSpec((_VB, _PACK * D), lambda c: (c, 0)),
        ],
        out_shape=[packed, packed],
    )

    @functools.partial(
        pl.kernel,
        mesh=mesh,
        out_type=jax.ShapeDtypeStruct((D * B,), jnp.float32),
        compiler_params=pltpu.CompilerParams(
            use_tc_tiling_on_sc=True, needs_layout_passes=False
        ),
        scratch_types=[
            pltpu.VMEM((b_per_w,), jnp.int32),
            pltpu.VMEM((b_per_w,), jnp.int32),
            pltpu.VMEM((b_per_w,), jnp.int32),
            pltpu.VMEM((b_per_w,), jnp.int32),
            pltpu.VMEM((b_per_w,), jnp.int32),
            pltpu.VMEM((b_per_w,), jnp.int32),
            pltpu.VMEM((_WAVE, _PACK * D), jnp.float32),
            pltpu.VMEM((_WAVE, _PACK * D), jnp.float32),
            pltpu.VMEM((D * b_per_w,), jnp.float32),
            pltpu.SemaphoreType.DMA,
            pltpu.SemaphoreType.DMA,
            pltpu.SemaphoreType.DMA,
        ],
    )
    def gmf(uid_hbm, iid_hbm, up_hbm, ip_hbm, out_hbm,
            uids_v, iids_v, urb_v, irb_v, ulq_v, ilq_v,
            uw_v, iw_v, prod_v, sem_u, sem_i, sem_o):
        wid = lax.axis_index("s") * NC + lax.axis_index("c")
        base = wid * b_per_w
        pltpu.sync_copy(uid_hbm.at[pl.ds(base, b_per_w)], uids_v)
        pltpu.sync_copy(iid_hbm.at[pl.ds(base, b_per_w)], iids_v)

        def prep(g, carry):
            s = pl.ds(g * L, L)
            u = uids_v[s]
            i = iids_v[s]
            urb_v[s] = lax.shift_left(lax.shift_right_logical(u, _SH + 2), _SH) + \
                lax.bitwise_and(u, _VB - 1)
            irb_v[s] = lax.shift_left(lax.shift_right_logical(i, _SH + 2), _SH) + \
                lax.bitwise_and(i, _VB - 1)
            ulq_v[s] = lax.shift_left(
                lax.bitwise_and(lax.shift_right_logical(u, _SH), _PACK - 1), 5)
            ilq_v[s] = lax.shift_left(
                lax.bitwise_and(lax.shift_right_logical(i, _SH), _PACK - 1), 5)
            return carry

        lax.fori_loop(0, b_per_w // L, prep, 0)

        row_iota = lax.iota(jnp.int32, L)
        for w in range(n_waves):
            cu = pltpu.async_copy(
                up_hbm.at[urb_v.at[pl.ds(w * _WAVE, _WAVE)]], uw_v, sem_u)
            ci = pltpu.async_copy(
                ip_hbm.at[irb_v.at[pl.ds(w * _WAVE, _WAVE)]], iw_v, sem_i)
            cu.wait()
            ci.wait()

            def select(d, carry, _w=w):
                for g in range(_WAVE // L):
                    rows = row_iota + (g * L)
                    sl = pl.ds(_w * _WAVE + g * L, L)
                    uv = plsc.load_gather(uw_v, [rows, ulq_v[sl] + d])
                    iv = plsc.load_gather(iw_v, [rows, ilq_v[sl] + d])
                    prod_v[pl.ds(d * b_per_w + _w * _WAVE + g * L, L)] = uv * iv
                return carry

            lax.fori_loop(0, D, select, 0)

        def flush(d, carry):
            pltpu.async_copy(
                prod_v.at[pl.ds(d * b_per_w, b_per_w)],
                out_hbm.at[pl.ds(d * B + base, b_per_w)],
                sem_o,
            )
            return carry

        lax.fori_loop(0, D, flush, 0)
        pltpu.make_async_copy(
            out_hbm.at[pl.ds(0, D * b_per_w)], prod_v, sem_o
        ).wait()

    def run(user_ids, item_ids, user_table, item_table):
        up, ip = detile(user_table.T, item_table.T)
        out1d = gmf(user_ids, item_ids, up, ip)
        return out1d.reshape(D, B).T

    return run


@jax.jit
def kernel(user_ids, item_ids, user_table, item_table):
    (B,) = user_ids.shape
    V, D = user_table.shape
    return _build(B, V, D)(user_ids, item_ids, user_table, item_table)


# speedup vs baseline: 35.5818x; 1.0223x over previous
import functools

import jax
import jax.numpy as jnp
from jax import lax
from jax.experimental import pallas as pl
from jax.experimental.pallas import tpu as pltpu
from jax.experimental.pallas import tpu_sc as plsc

_PACK = 4
_VB = 8192
_SH = _VB.bit_length() - 1
_WAVE = 128


def _detile_body(ut_ref, it_ref, uo_ref, io_ref):
    D = ut_ref.shape[0]
    row = lax.broadcasted_iota(jnp.int32, (_PACK * D, _PACK * D), 0)
    col = lax.broadcasted_iota(jnp.int32, (_PACK * D, _PACK * D), 1)
    eye = (row == col).astype(jnp.float32)
    for t_ref, out_ref in ((ut_ref, uo_ref), (it_ref, io_ref)):
        v = jnp.concatenate(
            [t_ref[:, t * _VB:(t + 1) * _VB] for t in range(_PACK)], axis=0)
        out_ref[...] = lax.dot_general(
            v, eye, (((0,), (0,)), ((), ())),
            precision=lax.Precision.DEFAULT,
            preferred_element_type=jnp.float32,
        )


@functools.lru_cache(maxsize=None)
def _build(B, V, D):
    info = plsc.get_sparse_core_info()
    NC, NS, L = info.num_cores, info.num_subcores, info.num_lanes
    NW = NC * NS
    assert B % NW == 0 and D == 32
    b_per_w = B // NW
    n_waves = b_per_w // _WAVE
    mesh = plsc.VectorSubcoreMesh(core_axis_name="c", subcore_axis_name="s")

    grid = (V + _PACK * _VB - 1) // (_PACK * _VB)
    packed_rows = grid * _VB
    packed = jax.ShapeDtypeStruct((packed_rows, _PACK * D), jnp.float32)
    detile = pl.pallas_call(
        _detile_body,
        grid=(grid,),
        in_specs=[
            pl.BlockSpec((D, _PACK * _VB), lambda c: (0, c)),
            pl.BlockSpec((D, _PACK * _VB), lambda c: (0, c)),
        ],
        out_specs=[
            pl.BlockSpec((_VB, _PACK * D), lambda c: (c, 0)),
            pl.BlockSpec((_VB, _PACK * D), lambda c: (c, 0)),
        ],
        out_shape=[packed, packed],
    )

    @functools.partial(
        pl.kernel,
        mesh=mesh,
        out_type=jax.ShapeDtypeStruct((D * B,), jnp.float32),
        compiler_params=pltpu.CompilerParams(
            use_tc_tiling_on_sc=True, needs_layout_passes=False
        ),
        scratch_types=[
            pltpu.VMEM((b_per_w,), jnp.int32),
            pltpu.VMEM((b_per_w,), jnp.int32),
            pltpu.VMEM((b_per_w,), jnp.int32),
            pltpu.VMEM((b_per_w,), jnp.int32),
            pltpu.VMEM((b_per_w,), jnp.int32),
            pltpu.VMEM((b_per_w,), jnp.int32),
            pltpu.VMEM((_WAVE, _PACK * D), jnp.float32),
            pltpu.VMEM((_WAVE, _PACK * D), jnp.float32),
            pltpu.VMEM((D * b_per_w,), jnp.float32),
            pltpu.SemaphoreType.DMA,
            pltpu.SemaphoreType.DMA,
            pltpu.SemaphoreType.DMA,
        ],
    )
    def gmf(uid_hbm, iid_hbm, up_hbm, ip_hbm, out_hbm,
            uids_v, iids_v, urb_v, irb_v, ulq_v, ilq_v,
            uw_v, iw_v, prod_v, sem_u, sem_i, sem_o):
        wid = lax.axis_index("s") * NC + lax.axis_index("c")
        base = wid * b_per_w
        pltpu.sync_copy(uid_hbm.at[pl.ds(base, b_per_w)], uids_v)
        pltpu.sync_copy(iid_hbm.at[pl.ds(base, b_per_w)], iids_v)

        def prep(g, carry):
            s = pl.ds(g * L, L)
            u = uids_v[s]
            i = iids_v[s]
            urb_v[s] = lax.shift_left(lax.shift_right_logical(u, _SH + 2), _SH) + \
                lax.bitwise_and(u, _VB - 1)
            irb_v[s] = lax.shift_left(lax.shift_right_logical(i, _SH + 2), _SH) + \
                lax.bitwise_and(i, _VB - 1)
            ulq_v[s] = lax.shift_left(
                lax.bitwise_and(lax.shift_right_logical(u, _SH), _PACK - 1), 5)
            ilq_v[s] = lax.shift_left(
                lax.bitwise_and(lax.shift_right_logical(i, _SH), _PACK - 1), 5)
            return carry

        lax.fori_loop(0, b_per_w // L, prep, 0)

        row_iota = lax.iota(jnp.int32, L)
        for w in range(n_waves):
            cu = pltpu.async_copy(
                up_hbm.at[urb_v.at[pl.ds(w * _WAVE, _WAVE)]], uw_v, sem_u)
            ci = pltpu.async_copy(
                ip_hbm.at[irb_v.at[pl.ds(w * _WAVE, _WAVE)]], iw_v, sem_i)
            cu.wait()
            ci.wait()

            def select(d, carry, _w=w):
                for g in range(_WAVE // L):
                    rows = row_iota + (g * L)
                    sl = pl.ds(_w * _WAVE + g * L, L)
                    uv = plsc.load_gather(uw_v, [rows, ulq_v[sl] + d])
                    iv = plsc.load_gather(iw_v, [rows, ilq_v[sl] + d])
                    prod_v[pl.ds(d * b_per_w + _w * _WAVE + g * L, L)] = uv * iv
                return carry

            lax.fori_loop(0, D, select, 0)

        def flush(d, carry):
            pltpu.async_copy(
                prod_v.at[pl.ds(d * b_per_w, b_per_w)],
                out_hbm.at[pl.ds(d * B + base, b_per_w)],
                sem_o,
            )
            return carry

        lax.fori_loop(0, D, flush, 0)
        pltpu.make_async_copy(
            out_hbm.at[pl.ds(0, D * b_per_w)], prod_v, sem_o
        ).wait()

    def run(user_ids, item_ids, user_table, item_table):
        up, ip = detile(user_table.T, item_table.T)
        out1d = gmf(user_ids, item_ids, up, ip)
        return out1d.reshape(D, B).T

    return run


@jax.jit
def kernel(user_ids, item_ids, user_table, item_table):
    (B,) = user_ids.shape
    V, D = user_table.shape
    return _build(B, V, D)(user_ids, item_ids, user_table, item_table)
